# baseline (speedup 1.0000x reference)
import functools

import jax
import jax.numpy as jnp
from jax import lax
from jax.experimental import pallas as pl
from jax.experimental.pallas import tpu as pltpu
from jax.experimental.pallas import tpu_sc as plsc

N = 10000
C = 128
E = 320000
NC = 2
NS = 16
NW = NC * NS
EPT = E // NW
CH = 128
NCH = 80
EPTP = NCH * CH
NPAD = 10112
ROWS = NPAD // NS
PADROW = N

_mesh = plsc.VectorSubcoreMesh(
    core_axis_name="c", subcore_axis_name="s", num_cores=NC, num_subcores=NS)


_DEG_FIRE = 4

@functools.partial(
    pl.kernel,
    out_type=jax.ShapeDtypeStruct((NC, NPAD, C), jnp.float32),
    mesh=_mesh,
    scratch_types=[
        pltpu.VMEM((NCH, CH), jnp.int32),
        pltpu.VMEM((CH, C), jnp.float32),
        pltpu.SemaphoreType.DMA,
        pltpu.VMEM_SHARED((NPAD, C), jnp.float32),
    ],
)
def _deg_kernel(dst_hbm, ones_hbm, zeros_hbm, out_hbm, di_all, ones_v, sem, acc):
    cid = lax.axis_index("c")
    sid = lax.axis_index("s")
    tid = sid * NC + cid
    pltpu.sync_copy(dst_hbm.at[tid], di_all)
    pltpu.sync_copy(ones_hbm, ones_v)
    pltpu.sync_copy(zeros_hbm, acc.at[pl.ds(sid * ROWS, ROWS)])
    plsc.subcore_barrier()

    def outer(j, carry):
        for b in range(_DEG_FIRE):
            pltpu.async_copy(ones_v, acc.at[di_all.at[j * _DEG_FIRE + b]],
                             sem, add=True)
        for b in range(_DEG_FIRE):
            pltpu.make_async_copy(ones_v, acc.at[di_all.at[j * _DEG_FIRE + b]],
                                  sem).wait()
        return carry

    lax.fori_loop(0, NCH // _DEG_FIRE, outer, 0)
    plsc.subcore_barrier()
    pltpu.sync_copy(acc.at[pl.ds(sid * ROWS, ROWS)],
                    out_hbm.at[cid, pl.ds(sid * ROWS, ROWS)])


@functools.partial(
    pl.kernel,
    out_type=jax.ShapeDtypeStruct((NC, NPAD, C), jnp.float32),
    mesh=_mesh,
    scratch_types=[
        pltpu.VMEM((NCH, CH), jnp.int32),
        pltpu.VMEM((CH,), jnp.int32),
        pltpu.VMEM((CH,), jnp.int32),
        pltpu.VMEM((CH, C), jnp.float32),
        pltpu.VMEM((CH, C), jnp.float32),
        pltpu.SemaphoreType.DMA,
        pltpu.SemaphoreType.DMA,
        pltpu.SemaphoreType.DMA,
        pltpu.SemaphoreType.DMA,
        pltpu.VMEM_SHARED((NPAD, C), jnp.float32),
    ],
)
def _scatter_kernel(src_hbm, dst_hbm, y_hbm, zeros_hbm, out_hbm,
                    di_all, si0, si1, rows0, rows1,
                    isem0, isem1, gsem0, gsem1, acc):
    cid = lax.axis_index("c")
    sid = lax.axis_index("s")
    tid = sid * NC + cid
    pltpu.sync_copy(dst_hbm.at[tid], di_all)
    pltpu.sync_copy(zeros_hbm, acc.at[pl.ds(sid * ROWS, ROWS)])
    plsc.subcore_barrier()

    si = (si0, si1)
    rows = (rows0, rows1)
    isem = (isem0, isem1)
    gsem = (gsem0, gsem1)

    pltpu.async_copy(src_hbm.at[tid, 0], si0, isem0)
    pltpu.async_copy(src_hbm.at[tid, 1], si1, isem1)
    pltpu.make_async_copy(src_hbm.at[tid, 0], si0, isem0).wait()
    pltpu.async_copy(y_hbm.at[si0], rows0, gsem0)

    def outer(j, carry):
        for b in range(2):
            i = j * 2 + b
            pltpu.make_async_copy(y_hbm.at[si[b]], rows[b], gsem[b]).wait()

            @pl.when(i + 2 < NCH)
            def _():
                pltpu.async_copy(src_hbm.at[tid, i + 2], si[b], isem[b])

            @pl.when(i + 1 < NCH)
            def _():
                pltpu.make_async_copy(src_hbm.at[tid, i + 1], si[1 - b],
                                      isem[1 - b]).wait()
                pltpu.async_copy(y_hbm.at[si[1 - b]], rows[1 - b], gsem[1 - b])

            pltpu.sync_copy(rows[b], acc.at[di_all.at[i]], add=True)
        return carry

    lax.fori_loop(0, NCH // 2, outer, 0)
    plsc.subcore_barrier()
    pltpu.sync_copy(acc.at[pl.ds(sid * ROWS, ROWS)],
                    out_hbm.at[cid, pl.ds(sid * ROWS, ROWS)])


def _bnmm_body(x_ref, g_ref, be_ref, w_ref, degp_ref, y_ref):
    x = x_ref[...]
    mean = jnp.mean(x, axis=0, keepdims=True)
    xc = x - mean
    var = jnp.mean(xc * xc, axis=0, keepdims=True)
    xh = xc * lax.rsqrt(var + 1e-5) * g_ref[...] + be_ref[...]
    xw = jnp.dot(xh, w_ref[...], preferred_element_type=jnp.float32)
    deg = degp_ref[0, 0:N, 0:1] + degp_ref[1, 0:N, 0:1] + 1.0
    y_ref[...] = xw * lax.rsqrt(deg)


_bnmm_call = pl.pallas_call(
    _bnmm_body, out_shape=jax.ShapeDtypeStruct((N, C), jnp.float32))


def _fin_body(p_ref, y_ref, degp_ref, b_ref, o_ref):
    deg = degp_ref[0, 0:N, 0:1] + degp_ref[1, 0:N, 0:1] + 1.0
    dis = lax.rsqrt(deg)
    s = p_ref[0, 0:N] + p_ref[1, 0:N] + y_ref[...]
    o_ref[...] = jnp.maximum(s * dis + b_ref[...], 0.0)


_fin_call = pl.pallas_call(
    _fin_body, out_shape=jax.ShapeDtypeStruct((N, C), jnp.float32))


def kernel(x, edge_index, bn_gamma, bn_beta, W, b):
    src = edge_index[0].astype(jnp.int32).reshape(NW, EPT)
    dst = edge_index[1].astype(jnp.int32).reshape(NW, EPT)
    srcp = jnp.pad(src, ((0, 0), (0, EPTP - EPT))).reshape(NW, NCH, CH)
    dstp = jnp.pad(dst, ((0, 0), (0, EPTP - EPT)),
                   constant_values=PADROW).reshape(NW, NCH, CH)
    ones_deg = jnp.ones((CH, C), jnp.float32)
    zeros_c = jnp.zeros((ROWS, C), jnp.float32)
    degp = _deg_kernel(dstp, ones_deg, zeros_c)
    y = _bnmm_call(x, bn_gamma.reshape(1, C), bn_beta.reshape(1, C), W, degp)
    p = _scatter_kernel(srcp, dstp, y, zeros_c)
    return _fin_call(p, y, degp, b.reshape(1, C))
